# Initial kernel scaffold; baseline (speedup 1.0000x reference)
import jax
import jax.numpy as jnp
from jax.experimental import pallas as pl


def kernel(x, edge_index, pre_W, pre_b, g1_W, g1_b, g2_W, g2_b, feat_emb, label_token, Wq, bq, Wk, bk, Wv, bv, Wo, bo, alpha, ffn_W1, ffn_b1, ffn_W2, ffn_b2, n1_g, n1_b, n2_g, n2_b, A, cls_W, cls_b):
    raise NotImplementedError("write your pallas kernel here")


# speedup vs baseline: 30.0397x; 30.0397x over previous
import functools
import math

import jax
import jax.numpy as jnp
from jax import lax
from jax.experimental import pallas as pl
from jax.experimental.pallas import tpu as pltpu
from jax.experimental.pallas import tpu_sc as plsc
from jax.scipy.linalg import block_diag

N = 10000
E = 320000
D_IN = 128
D = 64
D_EMB = 64
NH = 4
DK = 16
C = 40
S = D + 1
EPS = 1e-5

NC = 2
NS = 16
NW = NC * NS
EW = E // NW
CHUNK = 80
NCH = EW // CHUNK
N_PAD = 10240
RPT = N_PAD // NS


def _sc_mesh():
    return plsc.VectorSubcoreMesh(core_axis_name="c", subcore_axis_name="s")


def _sc_degree(dst_r):
    @functools.partial(
        pl.kernel,
        out_type=jax.ShapeDtypeStruct((NC, N_PAD, 16), jnp.float32),
        mesh=_sc_mesh(),
        compiler_params=pltpu.CompilerParams(use_tc_tiling_on_sc=False),
        scratch_types=[
            pltpu.VMEM((NCH, CHUNK), jnp.int32),
            pltpu.VMEM((CHUNK, 16), jnp.float32),
            pltpu.VMEM((RPT, 16), jnp.float32),
            pltpu.VMEM_SHARED((N_PAD, 16), jnp.float32),
        ],
    )
    def k(dst_hbm, out_hbm, dstv, ones_v, zb, acc):
        c = lax.axis_index("c")
        s = lax.axis_index("s")
        w = c * NS + s

        def fill_ones(i, carry):
            ones_v[i, :] = jnp.ones((16,), jnp.float32)
            return carry

        lax.fori_loop(0, CHUNK, fill_ones, 0)

        def fill_zero(i, carry):
            zb[i, :] = jnp.zeros((16,), jnp.float32)
            return carry

        lax.fori_loop(0, RPT, fill_zero, 0)

        pltpu.sync_copy(zb, acc.at[pl.ds(s * RPT, RPT)])
        plsc.subcore_barrier()

        pltpu.sync_copy(dst_hbm.at[w], dstv)

        def body(j, carry):
            pltpu.sync_copy(ones_v, acc.at[dstv.at[j]], add=True)
            return carry

        lax.fori_loop(0, NCH, body, 0)
        plsc.subcore_barrier()
        pltpu.sync_copy(acc.at[pl.ds(s * RPT, RPT)],
                        out_hbm.at[c, pl.ds(s * RPT, RPT)])

    return k(dst_r)


def _sc_aggregate(hs, src_r, dst_r):
    @functools.partial(
        pl.kernel,
        out_type=jax.ShapeDtypeStruct((NC, N_PAD, D), jnp.float32),
        mesh=_sc_mesh(),
        compiler_params=pltpu.CompilerParams(use_tc_tiling_on_sc=False),
        scratch_types=[
            pltpu.VMEM((NCH, CHUNK), jnp.int32),
            pltpu.VMEM((NCH, CHUNK), jnp.int32),
            pltpu.VMEM((CHUNK, D), jnp.float32),
            pltpu.VMEM((RPT, D), jnp.float32),
            pltpu.VMEM_SHARED((N_PAD, D), jnp.float32),
            pltpu.SemaphoreType.DMA,
        ],
    )
    def k(hs_hbm, src_hbm, dst_hbm, out_hbm, srcv, dstv, rows, zb, acc, sem):
        c = lax.axis_index("c")
        s = lax.axis_index("s")
        w = c * NS + s

        def fill_zero(i, carry):
            for q in range(D // 16):
                zb[i, pl.ds(q * 16, 16)] = jnp.zeros((16,), jnp.float32)
            return carry

        lax.fori_loop(0, RPT, fill_zero, 0)
        pltpu.sync_copy(zb, acc.at[pl.ds(s * RPT, RPT)])
        plsc.subcore_barrier()

        pltpu.sync_copy(src_hbm.at[w], srcv)
        pltpu.sync_copy(dst_hbm.at[w], dstv)

        def body(j, carry):
            pltpu.async_copy(hs_hbm.at[srcv.at[j]], rows, sem).wait()
            pltpu.sync_copy(rows, acc.at[dstv.at[j]], add=True)
            return carry

        lax.fori_loop(0, NCH, body, 0)
        plsc.subcore_barrier()
        pltpu.sync_copy(acc.at[pl.ds(s * RPT, RPT)],
                        out_hbm.at[c, pl.ds(s * RPT, RPT)])

    return k(hs, src_r, dst_r)


_B = 1000


def _dinv_from(degp_ref):
    deg = degp_ref[0, :, 0:1] + degp_ref[1, :, 0:1] + 1.0
    return lax.rsqrt(deg)


def _gelu(v):
    return 0.5 * v * (1.0 + lax.erf(v * (1.0 / math.sqrt(2.0))))


def _full2d(shape):
    return pl.BlockSpec(shape, lambda i: (0, 0))


def _tc_pre(x, degp, pre_W, pre_b, g1_W):
    def body(x_ref, degp_ref, pw_ref, pb_ref, gw_ref, out_ref):
        dinv = _dinv_from(degp_ref)
        h0 = jnp.dot(x_ref[...], pw_ref[...],
                     preferred_element_type=jnp.float32) + pb_ref[...]
        t1 = jnp.dot(h0, gw_ref[...], preferred_element_type=jnp.float32)
        out_ref[...] = t1 * dinv

    return pl.pallas_call(
        body,
        grid=(N // _B,),
        in_specs=[
            pl.BlockSpec((_B, D_IN), lambda i: (i, 0)),
            pl.BlockSpec((NC, _B, 16), lambda i: (0, i, 0)),
            _full2d((D_IN, D)),
            _full2d((1, D)),
            _full2d((D, D)),
        ],
        out_specs=pl.BlockSpec((_B, D), lambda i: (i, 0)),
        out_shape=jax.ShapeDtypeStruct((N, D), jnp.float32),
    )(x, degp, pre_W, pre_b.reshape(1, D), g1_W)


def _tc_mid(acc1p, hs1, degp, g1_b, g2_W):
    def body(a_ref, hs_ref, degp_ref, gb_ref, gw_ref, out_ref):
        dinv = _dinv_from(degp_ref)
        out1 = dinv * (a_ref[0] + a_ref[1] + hs_ref[...]) + gb_ref[...]
        h1 = _gelu(out1)
        t2 = jnp.dot(h1, gw_ref[...], preferred_element_type=jnp.float32)
        out_ref[...] = t2 * dinv

    return pl.pallas_call(
        body,
        grid=(N // _B,),
        in_specs=[
            pl.BlockSpec((NC, _B, D), lambda i: (0, i, 0)),
            pl.BlockSpec((_B, D), lambda i: (i, 0)),
            pl.BlockSpec((NC, _B, 16), lambda i: (0, i, 0)),
            _full2d((1, D)),
            _full2d((D, D)),
        ],
        out_specs=pl.BlockSpec((_B, D), lambda i: (i, 0)),
        out_shape=jax.ShapeDtypeStruct((N, D), jnp.float32),
    )(acc1p, hs1, degp, g1_b.reshape(1, D), g2_W)


def _tc_head(acc2p, hs2, degp, g2_b, consts):
    (s_f, a_coef, b_coef, s64, qv, v64, xn64, Pv_bd, Wo, bo,
     n2_g, n2_b, ffn_W1, ffn_b1, ffn_W2, ffn_b2, alpha2d, cls_W, cls_b) = consts

    def body(a_ref, hs_ref, degp_ref, gb_ref, sf_ref, ac_ref, bc_ref,
             s64_ref, qv_ref, v64_ref, xn_ref, pv_ref, wo_ref, bo_ref,
             n2g_ref, n2b_ref, w1_ref, b1_ref, w2_ref, b2_ref, al_ref,
             cw_ref, cb_ref, out_ref):
        dinv = _dinv_from(degp_ref)
        h2 = dinv * (a_ref[0] + a_ref[1] + hs_ref[...]) + gb_ref[...]

        mu = jnp.mean(h2, axis=1, keepdims=True)
        zc = h2 - mu
        var = jnp.mean(zc * zc, axis=1, keepdims=True)
        zn = zc * lax.rsqrt(var + EPS)
        r = zn * lax.rsqrt(zn * zn * sf_ref[...] + EPS)

        wr_parts = []
        extra_parts = []
        for h in range(NH):
            sch = r * ac_ref[h:h + 1, :] + bc_ref[h:h + 1, :]
            s64h = s64_ref[0:1, h:h + 1]
            mh = jnp.maximum(jnp.max(sch, axis=1, keepdims=True), s64h)
            eh = jnp.exp(sch - mh)
            elh = jnp.exp(s64h - mh)
            inv_den = 1.0 / (jnp.sum(eh, axis=1, keepdims=True) + elh)
            wlh = elh * inv_den
            wr_parts.append((eh * inv_den) * r)
            extra_parts.append((1.0 - wlh) * qv_ref[0:1, h * DK:(h + 1) * DK]
                               + wlh * v64_ref[0:1, h * DK:(h + 1) * DK])
        WR = jnp.concatenate(wr_parts, axis=1)
        attn = (jnp.dot(WR, pv_ref[...], preferred_element_type=jnp.float32)
                + jnp.concatenate(extra_parts, axis=1))

        o = jnp.dot(attn, wo_ref[...],
                    preferred_element_type=jnp.float32) + bo_ref[...]
        mu2 = jnp.mean(o, axis=1, keepdims=True)
        oc = o - mu2
        var2 = jnp.mean(oc * oc, axis=1, keepdims=True)
        on2 = oc * lax.rsqrt(var2 + EPS) * n2g_ref[...] + n2b_ref[...]
        f = _gelu(jnp.dot(on2, w1_ref[...],
                          preferred_element_type=jnp.float32) + b1_ref[...])
        f2 = jnp.dot(f, w2_ref[...],
                     preferred_element_type=jnp.float32) + b2_ref[...]
        hh = o + f2
        zf = al_ref[...] * hh + xn_ref[...]
        out_ref[...] = jnp.dot(zf, cw_ref[...],
                               preferred_element_type=jnp.float32) + cb_ref[...]

    return pl.pallas_call(
        body,
        grid=(N // _B,),
        in_specs=[
            pl.BlockSpec((NC, _B, D), lambda i: (0, i, 0)),
            pl.BlockSpec((_B, D), lambda i: (i, 0)),
            pl.BlockSpec((NC, _B, 16), lambda i: (0, i, 0)),
            _full2d((1, D)),
            _full2d((1, D)),
            _full2d((NH, D)),
            _full2d((NH, D)),
            _full2d((1, NH)),
            _full2d((1, D_EMB)),
            _full2d((1, D_EMB)),
            _full2d((1, D_EMB)),
            _full2d((NH * D, D_EMB)),
            _full2d((D_EMB, D_EMB)),
            _full2d((1, D_EMB)),
            _full2d((1, D_EMB)),
            _full2d((1, D_EMB)),
            _full2d((D_EMB, 2 * D_EMB)),
            _full2d((1, 2 * D_EMB)),
            _full2d((2 * D_EMB, D_EMB)),
            _full2d((1, D_EMB)),
            _full2d((1, 1)),
            _full2d((D_EMB, C)),
            _full2d((1, C)),
        ],
        out_specs=pl.BlockSpec((_B, C), lambda i: (i, 0)),
        out_shape=jax.ShapeDtypeStruct((N, C), jnp.float32),
    )(acc2p, hs2, degp, g2_b.reshape(1, D), s_f, a_coef, b_coef, s64, qv,
      v64, xn64, Pv_bd, Wo, bo.reshape(1, D_EMB), n2_g.reshape(1, D_EMB),
      n2_b.reshape(1, D_EMB), ffn_W1, ffn_b1.reshape(1, 2 * D_EMB), ffn_W2,
      ffn_b2.reshape(1, D_EMB), alpha2d, cls_W, cls_b.reshape(1, C))


def _head_consts(feat_emb, label_token, Wq, bq, Wk, bk, Wv, bv, n1_g, n1_b,
                 A, alpha):
    mask = jnp.ones((S, S), jnp.float32)
    mask = mask.at[D:, :].set(0.0)
    mask = mask * (1.0 - jnp.eye(S, dtype=jnp.float32))
    A_nd = A * mask
    cm = jnp.abs(A_nd).T
    cmax = jnp.max(cm)
    cm = jnp.where(cmax > 1e-6, cm / jnp.where(cmax > 1e-6, cmax, 1.0),
                   cm + 0.001)
    cm = (cm * (1.0 - jnp.eye(S, dtype=jnp.float32))
          + jnp.eye(S, dtype=jnp.float32))
    dag64 = jnp.log(cm + 1e-9)[S - 1]

    m_f = jnp.mean(feat_emb, axis=1)
    s_f = jnp.var(feat_emb, axis=1)
    gu = (feat_emb - m_f[:, None]) * n1_g[None, :]

    lab = label_token.reshape(D_EMB)
    lab_n = (lab - jnp.mean(lab)) * lax.rsqrt(jnp.var(lab) + EPS)
    xn64 = lab_n * n1_g + n1_b

    qrow = xn64 @ Wq + bq
    Pk = gu @ Wk
    qk = n1_b @ Wk + bk
    k64 = xn64 @ Wk + bk
    Pv = gu @ Wv
    qv = n1_b @ Wv + bv
    v64 = xn64 @ Wv + bv

    scale = 1.0 / math.sqrt(DK)
    qh = qrow.reshape(NH, DK)
    a_coef = jnp.einsum('hd,ihd->hi', qh, Pk.reshape(D, NH, DK)) * scale
    b_coef = ((qh * qk.reshape(NH, DK)).sum(-1)[:, None] * scale
              + dag64[None, :D])
    s64 = ((qh * k64.reshape(NH, DK)).sum(-1) * scale + dag64[D]).reshape(1, NH)

    Pvh = Pv.reshape(D, NH, DK)
    Pv_bd = block_diag(*[Pvh[:, h, :] for h in range(NH)])

    return (s_f.reshape(1, D), a_coef, b_coef, s64, qv.reshape(1, D_EMB),
            v64.reshape(1, D_EMB), xn64.reshape(1, D_EMB), Pv_bd,
            alpha.reshape(1, 1))


def kernel(x, edge_index, pre_W, pre_b, g1_W, g1_b, g2_W, g2_b, feat_emb,
           label_token, Wq, bq, Wk, bk, Wv, bv, Wo, bo, alpha, ffn_W1,
           ffn_b1, ffn_W2, ffn_b2, n1_g, n1_b, n2_g, n2_b, A, cls_W, cls_b):
    src_r = edge_index[0].reshape(NW, NCH, CHUNK)
    dst_r = edge_index[1].reshape(NW, NCH, CHUNK)

    degp = _sc_degree(dst_r)
    hs1 = _tc_pre(x, degp, pre_W, pre_b, g1_W)
    acc1p = _sc_aggregate(hs1, src_r, dst_r)
    hs2 = _tc_mid(acc1p, hs1, degp, g1_b, g2_W)
    acc2p = _sc_aggregate(hs2, src_r, dst_r)

    (s_f, a_coef, b_coef, s64, qv, v64, xn64, Pv_bd, alpha2d) = _head_consts(
        feat_emb, label_token, Wq, bq, Wk, bk, Wv, bv, n1_g, n1_b, A, alpha)
    consts = (s_f, a_coef, b_coef, s64, qv, v64, xn64, Pv_bd, Wo, bo,
              n2_g, n2_b, ffn_W1, ffn_b1, ffn_W2, ffn_b2, alpha2d,
              cls_W, cls_b)
    return _tc_head(acc2p, hs2, degp, g2_b, consts)
